# baseline (speedup 1.0000x reference)
import functools

import jax
import jax.numpy as jnp
from jax import lax
from jax.experimental import pallas as pl
from jax.experimental.pallas import tpu as pltpu
from jax.experimental.pallas import tpu_sc as plsc

DIM = 1024
HEADS = 4
NUM_KEYS = 256
TOPK = 16
DIM_HEAD = 128
NCHUNK = 2 * HEADS

TOK_BLOCK = 256

_NEG = float("-inf")


def _topk_t(d):
    pos = lax.broadcasted_iota(jnp.int32, d.shape, 0)
    ms, js = [], []
    for _ in range(TOPK):
        m = jnp.max(d, axis=0, keepdims=True)
        j = jnp.min(jnp.where(d == m, pos, 1 << 30), axis=0, keepdims=True)
        ms.append(m)
        js.append(j)
        d = jnp.where(pos == j, _NEG, d)
    return jnp.concatenate(ms, axis=0), jnp.concatenate(js, axis=0)


def _route_body(xt_ref, wq_ref, g_ref, b_ref, keys_ref, w_ref, vi_ref):
    qt = jnp.dot(wq_ref[...], xt_ref[...],
                 preferred_element_type=jnp.float32)
    gamma = g_ref[...]
    beta = b_ref[...]

    def stage1(c):
        qc = qt[c * DIM_HEAD:(c + 1) * DIM_HEAD, :]
        mean = jnp.mean(qc, axis=0, keepdims=True)
        cen = qc - mean
        var = jnp.mean(cen * cen, axis=0, keepdims=True)
        qn = cen * lax.rsqrt(var + 1e-5) * gamma + beta
        d = jnp.dot(keys_ref[c], qn,
                    preferred_element_type=jnp.float32)
        return _topk_t(d)

    for h in range(HEADS):
        sx, ixp = stage1(h)
        sy, iyp = stage1(HEADS + h)
        comb = jnp.concatenate(
            [sx[j:j + 1, :] + sy for j in range(TOPK)], axis=0)
        fs, fp = _topk_t(comb)
        jx = fp >> 4
        jy = fp & 15
        vix = jnp.zeros(fp.shape, jnp.int32)
        viy = jnp.zeros(fp.shape, jnp.int32)
        for cc in range(TOPK):
            vix = jnp.where(jx == cc, ixp[cc:cc + 1, :], vix)
            viy = jnp.where(jy == cc, iyp[cc:cc + 1, :], viy)
        e = jnp.exp(fs - fs[0:1, :])
        w = e / jnp.sum(e, axis=0, keepdims=True)
        w_ref[h * TOPK:(h + 1) * TOPK, :] = w
        vi_ref[h * TOPK:(h + 1) * TOPK, :] = vix * NUM_KEYS + viy


def _route(xt, wq, gamma, beta, kmat):
    t = xt.shape[1]
    grid = (t // TOK_BLOCK,)
    return pl.pallas_call(
        _route_body,
        grid=grid,
        in_specs=[
            pl.BlockSpec((DIM, TOK_BLOCK), lambda i: (0, i)),
            pl.BlockSpec((2 * HEADS * DIM_HEAD, DIM), lambda i: (0, 0)),
            pl.BlockSpec((DIM_HEAD, 1), lambda i: (0, 0)),
            pl.BlockSpec((DIM_HEAD, 1), lambda i: (0, 0)),
            pl.BlockSpec((NCHUNK, NUM_KEYS, DIM_HEAD), lambda i: (0, 0, 0)),
        ],
        out_specs=[
            pl.BlockSpec((HEADS * TOPK, TOK_BLOCK), lambda i: (0, i)),
            pl.BlockSpec((HEADS * TOPK, TOK_BLOCK), lambda i: (0, i)),
        ],
        out_shape=[
            jax.ShapeDtypeStruct((HEADS * TOPK, t), jnp.float32),
            jax.ShapeDtypeStruct((HEADS * TOPK, t), jnp.int32),
        ],
    )(xt, wq, gamma, beta, kmat)


try:
    _SC_INFO = plsc.get_sparse_core_info()
    _NC = _SC_INFO.num_cores
    _NS = _SC_INFO.num_subcores
except Exception:
    _NC, _NS = 2, 16
_NW = _NC * _NS
_HALF = (HEADS * TOPK) // 2


_OBATCH = 8


def _bag_body(values_hbm, vi_hbm, w_hbm, out_hbm,
              vi_v, w_v, rows_v, out_v, idx0, idx1, sem0, sem1):
    ntok = vi_hbm.shape[0] // _NW
    wid = lax.axis_index("s") * _NC + lax.axis_index("c")
    base = pl.multiple_of(wid * ntok, ntok)
    pltpu.sync_copy(vi_hbm.at[pl.ds(base, ntok)], vi_v)
    pltpu.sync_copy(w_hbm.at[pl.ds(base, ntok)], w_v)
    idx_bufs = [idx0, idx1]

    def stage_idx(tok, half, buf):
        ib = idx_bufs[buf]
        ib[pl.ds(0, 16)] = vi_v[tok, pl.ds(half * _HALF, 16)]
        ib[pl.ds(16, 16)] = vi_v[tok, pl.ds(half * _HALF + 16, 16)]

    def copy(tok, half, buf, sem):
        return pltpu.make_async_copy(values_hbm.at[idx_bufs[buf]],
                                     rows_v.at[buf], sem)

    def accumulate(tok, half, buf, tm):
        for g in range(DIM // 128):
            gb = g * 128

            def rg_body(rg, accs):
                wv = w_v[tok, pl.ds(half * _HALF + rg * 16, 16)]
                for j in range(16):
                    wr = jnp.full((16,), wv[j], jnp.float32)
                    r = rg * 16 + j
                    accs = tuple(
                        accs[i] + rows_v[buf, r, pl.ds(gb + i * 16, 16)] * wr
                        for i in range(8))
                return accs

            accs = lax.fori_loop(
                0, _HALF // 16, rg_body,
                tuple(jnp.zeros((16,), jnp.float32) for _ in range(8)))
            for i in range(8):
                sl = pl.ds(gb + i * 16, 16)
                if half == 0:
                    out_v[tm, sl] = accs[i]
                else:
                    out_v[tm, sl] = out_v[tm, sl] + accs[i]

    stage_idx(0, 0, 0)
    copy(0, 0, 0, sem0).start()

    def tok_body(tok, _):
        tm = tok % _OBATCH
        stage_idx(tok, 1, 1)
        copy(tok, 1, 1, sem1).start()
        copy(tok, 0, 0, sem0).wait()
        accumulate(tok, 0, 0, tm)

        @pl.when(tok < ntok - 1)
        def _():
            stage_idx(tok + 1, 0, 0)
            copy(tok + 1, 0, 0, sem0).start()

        copy(tok, 1, 1, sem1).wait()
        accumulate(tok, 1, 1, tm)

        @pl.when(tm == _OBATCH - 1)
        def _():
            start = pl.multiple_of(base + tok - (_OBATCH - 1), _OBATCH)
            pltpu.sync_copy(out_v, out_hbm.at[pl.ds(start, _OBATCH)])
        return 0

    lax.fori_loop(0, ntok, tok_body, 0)


def _bag(values, vi, w, ntok):
    mesh = plsc.VectorSubcoreMesh(core_axis_name="c", subcore_axis_name="s")
    per = ntok // _NW
    kern = pl.kernel(
        _bag_body,
        out_type=jax.ShapeDtypeStruct((ntok, DIM), jnp.float32),
        mesh=mesh,
        scratch_types=[
            pltpu.VMEM((per, HEADS * TOPK), jnp.int32),
            pltpu.VMEM((per, HEADS * TOPK), jnp.float32),
            pltpu.VMEM((2, _HALF, DIM), jnp.float32),
            pltpu.VMEM((_OBATCH, DIM), jnp.float32),
            pltpu.VMEM((_HALF,), jnp.int32),
            pltpu.VMEM((_HALF,), jnp.int32),
            pltpu.SemaphoreType.DMA,
            pltpu.SemaphoreType.DMA,
        ],
    )
    return kern(values, vi, w)


def kernel(x, Wq, ln_gamma, ln_beta, keys_p, values):
    b, t, _ = x.shape
    xt = x.reshape(t, DIM).T
    kmat = jnp.transpose(keys_p, (2, 0, 1, 3)).reshape(
        NCHUNK, NUM_KEYS, DIM_HEAD)
    gamma = ln_gamma.reshape(DIM_HEAD, 1)
    beta = ln_beta.reshape(DIM_HEAD, 1)
    nch = 8
    tc = t // nch
    outs = []
    for i in range(nch):
        wt, vit = _route(xt[:, i * tc:(i + 1) * tc], Wq, gamma, beta, kmat)
        outs.append(_bag(values, vit.T, wt.T, tc))
    out = jnp.concatenate(outs, axis=0)
    return out.reshape(b, t, DIM)


# speedup vs baseline: 1.0835x; 1.0835x over previous
import functools

import jax
import jax.numpy as jnp
from jax import lax
from jax.experimental import pallas as pl
from jax.experimental.pallas import tpu as pltpu
from jax.experimental.pallas import tpu_sc as plsc

DIM = 1024
HEADS = 4
NUM_KEYS = 256
TOPK = 16
DIM_HEAD = 128
NCHUNK = 2 * HEADS

TOK_BLOCK = 512

_NEG = float("-inf")


def _topk_t(d):
    pos = lax.broadcasted_iota(jnp.int32, d.shape, 0)
    ms, js = [], []
    for _ in range(TOPK):
        m = jnp.max(d, axis=0, keepdims=True)
        j = jnp.min(jnp.where(d == m, pos, 1 << 30), axis=0, keepdims=True)
        ms.append(m)
        js.append(j)
        d = jnp.where(pos == j, _NEG, d)
    return jnp.concatenate(ms, axis=0), jnp.concatenate(js, axis=0)


def _route_body(xt_ref, wq_ref, g_ref, b_ref, keys_ref, w_ref, vi_ref):
    qt = jnp.dot(wq_ref[...], xt_ref[...],
                 preferred_element_type=jnp.float32)
    gamma = g_ref[...]
    beta = b_ref[...]

    def stage1(c):
        qc = qt[c * DIM_HEAD:(c + 1) * DIM_HEAD, :]
        mean = jnp.mean(qc, axis=0, keepdims=True)
        cen = qc - mean
        var = jnp.mean(cen * cen, axis=0, keepdims=True)
        qn = cen * lax.rsqrt(var + 1e-5) * gamma + beta
        d = jnp.dot(keys_ref[c], qn,
                    preferred_element_type=jnp.float32)
        return _topk_t(d)

    for h in range(HEADS):
        sx, ixp = stage1(h)
        sy, iyp = stage1(HEADS + h)
        comb = jnp.concatenate(
            [sx[j:j + 1, :] + sy for j in range(TOPK)], axis=0)
        fs, fp = _topk_t(comb)
        jx = fp >> 4
        jy = fp & 15
        vix = jnp.zeros(fp.shape, jnp.int32)
        viy = jnp.zeros(fp.shape, jnp.int32)
        for cc in range(TOPK):
            vix = jnp.where(jx == cc, ixp[cc:cc + 1, :], vix)
            viy = jnp.where(jy == cc, iyp[cc:cc + 1, :], viy)
        e = jnp.exp(fs - fs[0:1, :])
        w = e / jnp.sum(e, axis=0, keepdims=True)
        w_ref[h * TOPK:(h + 1) * TOPK, :] = w
        vi_ref[h * TOPK:(h + 1) * TOPK, :] = vix * NUM_KEYS + viy


def _route(xt, wq, gamma, beta, kmat):
    t = xt.shape[1]
    grid = (t // TOK_BLOCK,)
    return pl.pallas_call(
        _route_body,
        grid=grid,
        in_specs=[
            pl.BlockSpec((DIM, TOK_BLOCK), lambda i: (0, i)),
            pl.BlockSpec((2 * HEADS * DIM_HEAD, DIM), lambda i: (0, 0)),
            pl.BlockSpec((DIM_HEAD, 1), lambda i: (0, 0)),
            pl.BlockSpec((DIM_HEAD, 1), lambda i: (0, 0)),
            pl.BlockSpec((NCHUNK, NUM_KEYS, DIM_HEAD), lambda i: (0, 0, 0)),
        ],
        out_specs=[
            pl.BlockSpec((HEADS * TOPK, TOK_BLOCK), lambda i: (0, i)),
            pl.BlockSpec((HEADS * TOPK, TOK_BLOCK), lambda i: (0, i)),
        ],
        out_shape=[
            jax.ShapeDtypeStruct((HEADS * TOPK, t), jnp.float32),
            jax.ShapeDtypeStruct((HEADS * TOPK, t), jnp.int32),
        ],
    )(xt, wq, gamma, beta, kmat)


try:
    _SC_INFO = plsc.get_sparse_core_info()
    _NC = _SC_INFO.num_cores
    _NS = _SC_INFO.num_subcores
except Exception:
    _NC, _NS = 2, 16
_NW = _NC * _NS
_HALF = (HEADS * TOPK) // 2


_OBATCH = 8


def _bag_body(values_hbm, vi_hbm, w_hbm, out_hbm,
              vi_v, w_v, rows_v, out_v, idx0, idx1, sem0, sem1):
    ntok = vi_hbm.shape[0] // _NW
    wid = lax.axis_index("s") * _NC + lax.axis_index("c")
    base = pl.multiple_of(wid * ntok, ntok)
    pltpu.sync_copy(vi_hbm.at[pl.ds(base, ntok)], vi_v)
    pltpu.sync_copy(w_hbm.at[pl.ds(base, ntok)], w_v)
    idx_bufs = [idx0, idx1]

    def stage_idx(tok, half, buf):
        ib = idx_bufs[buf]
        ib[pl.ds(0, 16)] = vi_v[tok, pl.ds(half * _HALF, 16)]
        ib[pl.ds(16, 16)] = vi_v[tok, pl.ds(half * _HALF + 16, 16)]

    def copy(tok, half, buf, sem):
        return pltpu.make_async_copy(values_hbm.at[idx_bufs[buf]],
                                     rows_v.at[buf], sem)

    def accumulate(tok, half, buf, tm):
        for g in range(DIM // 128):
            gb = g * 128

            def rg_body(rg, accs):
                wv = w_v[tok, pl.ds(half * _HALF + rg * 16, 16)]
                for j in range(16):
                    wr = jnp.full((16,), wv[j], jnp.float32)
                    r = rg * 16 + j
                    accs = tuple(
                        accs[i] + rows_v[buf, r, pl.ds(gb + i * 16, 16)] * wr
                        for i in range(8))
                return accs

            accs = lax.fori_loop(
                0, _HALF // 16, rg_body,
                tuple(jnp.zeros((16,), jnp.float32) for _ in range(8)))
            for i in range(8):
                sl = pl.ds(gb + i * 16, 16)
                if half == 0:
                    out_v[tm, sl] = accs[i]
                else:
                    out_v[tm, sl] = out_v[tm, sl] + accs[i]

    stage_idx(0, 0, 0)
    copy(0, 0, 0, sem0).start()

    def tok_body(tok, _):
        tm = tok % _OBATCH
        stage_idx(tok, 1, 1)
        copy(tok, 1, 1, sem1).start()
        copy(tok, 0, 0, sem0).wait()
        accumulate(tok, 0, 0, tm)

        @pl.when(tok < ntok - 1)
        def _():
            stage_idx(tok + 1, 0, 0)
            copy(tok + 1, 0, 0, sem0).start()

        copy(tok, 1, 1, sem1).wait()
        accumulate(tok, 1, 1, tm)

        @pl.when(tm == _OBATCH - 1)
        def _():
            start = pl.multiple_of(base + tok - (_OBATCH - 1), _OBATCH)
            pltpu.sync_copy(out_v, out_hbm.at[pl.ds(start, _OBATCH)])
        return 0

    lax.fori_loop(0, ntok, tok_body, 0)


def _bag(values, vi, w, ntok):
    mesh = plsc.VectorSubcoreMesh(core_axis_name="c", subcore_axis_name="s")
    per = ntok // _NW
    kern = pl.kernel(
        _bag_body,
        out_type=jax.ShapeDtypeStruct((ntok, DIM), jnp.float32),
        mesh=mesh,
        scratch_types=[
            pltpu.VMEM((per, HEADS * TOPK), jnp.int32),
            pltpu.VMEM((per, HEADS * TOPK), jnp.float32),
            pltpu.VMEM((2, _HALF, DIM), jnp.float32),
            pltpu.VMEM((_OBATCH, DIM), jnp.float32),
            pltpu.VMEM((_HALF,), jnp.int32),
            pltpu.VMEM((_HALF,), jnp.int32),
            pltpu.SemaphoreType.DMA,
            pltpu.SemaphoreType.DMA,
        ],
    )
    return kern(values, vi, w)


def kernel(x, Wq, ln_gamma, ln_beta, keys_p, values):
    b, t, _ = x.shape
    xt = x.reshape(t, DIM).T
    kmat = jnp.transpose(keys_p, (2, 0, 1, 3)).reshape(
        NCHUNK, NUM_KEYS, DIM_HEAD)
    gamma = ln_gamma.reshape(DIM_HEAD, 1)
    beta = ln_beta.reshape(DIM_HEAD, 1)
    nch = 4
    tc = t // nch
    outs = []
    for i in range(nch):
        wt, vit = _route(xt[:, i * tc:(i + 1) * tc], Wq, gamma, beta, kmat)
        outs.append(_bag(values, vit.T, wt.T, tc))
    out = jnp.concatenate(outs, axis=0)
    return out.reshape(b, t, DIM)


# speedup vs baseline: 1.0867x; 1.0030x over previous
import functools

import jax
import jax.numpy as jnp
from jax import lax
from jax.experimental import pallas as pl
from jax.experimental.pallas import tpu as pltpu
from jax.experimental.pallas import tpu_sc as plsc

DIM = 1024
HEADS = 4
NUM_KEYS = 256
TOPK = 16
DIM_HEAD = 128
NCHUNK = 2 * HEADS

TOK_BLOCK = 256

_NEG = float("-inf")


def _topk_t(d):
    pos = lax.broadcasted_iota(jnp.int32, d.shape, 0)
    ms, js = [], []
    for _ in range(TOPK):
        m = jnp.max(d, axis=0, keepdims=True)
        j = jnp.min(jnp.where(d == m, pos, 1 << 30), axis=0, keepdims=True)
        ms.append(m)
        js.append(j)
        d = jnp.where(pos == j, _NEG, d)
    return jnp.concatenate(ms, axis=0), jnp.concatenate(js, axis=0)


def _route_body(xt_ref, wq_ref, g_ref, b_ref, keys_ref, w_ref, vi_ref):
    qt = jnp.dot(wq_ref[...], xt_ref[...],
                 preferred_element_type=jnp.float32)
    gamma = g_ref[...]
    beta = b_ref[...]

    def stage1(c):
        qc = qt[c * DIM_HEAD:(c + 1) * DIM_HEAD, :]
        mean = jnp.mean(qc, axis=0, keepdims=True)
        cen = qc - mean
        var = jnp.mean(cen * cen, axis=0, keepdims=True)
        qn = cen * lax.rsqrt(var + 1e-5) * gamma + beta
        d = jnp.dot(keys_ref[c], qn,
                    preferred_element_type=jnp.float32)
        return _topk_t(d)

    for h in range(HEADS):
        sx, ixp = stage1(h)
        sy, iyp = stage1(HEADS + h)
        comb = jnp.concatenate(
            [sx[j:j + 1, :] + sy for j in range(TOPK)], axis=0)
        fs, fp = _topk_t(comb)
        jx = fp >> 4
        jy = fp & 15
        vix = jnp.zeros(fp.shape, jnp.int32)
        viy = jnp.zeros(fp.shape, jnp.int32)
        for cc in range(TOPK):
            vix = jnp.where(jx == cc, ixp[cc:cc + 1, :], vix)
            viy = jnp.where(jy == cc, iyp[cc:cc + 1, :], viy)
        e = jnp.exp(fs - fs[0:1, :])
        w = e / jnp.sum(e, axis=0, keepdims=True)
        w_ref[h * TOPK:(h + 1) * TOPK, :] = w
        vi_ref[h * TOPK:(h + 1) * TOPK, :] = vix * NUM_KEYS + viy


def _route(xt, wq, gamma, beta, kmat):
    t = xt.shape[1]
    grid = (t // TOK_BLOCK,)
    return pl.pallas_call(
        _route_body,
        grid=grid,
        in_specs=[
            pl.BlockSpec((DIM, TOK_BLOCK), lambda i: (0, i)),
            pl.BlockSpec((2 * HEADS * DIM_HEAD, DIM), lambda i: (0, 0)),
            pl.BlockSpec((DIM_HEAD, 1), lambda i: (0, 0)),
            pl.BlockSpec((DIM_HEAD, 1), lambda i: (0, 0)),
            pl.BlockSpec((NCHUNK, NUM_KEYS, DIM_HEAD), lambda i: (0, 0, 0)),
        ],
        out_specs=[
            pl.BlockSpec((HEADS * TOPK, TOK_BLOCK), lambda i: (0, i)),
            pl.BlockSpec((HEADS * TOPK, TOK_BLOCK), lambda i: (0, i)),
        ],
        out_shape=[
            jax.ShapeDtypeStruct((HEADS * TOPK, t), jnp.float32),
            jax.ShapeDtypeStruct((HEADS * TOPK, t), jnp.int32),
        ],
    )(xt, wq, gamma, beta, kmat)


try:
    _SC_INFO = plsc.get_sparse_core_info()
    _NC = _SC_INFO.num_cores
    _NS = _SC_INFO.num_subcores
except Exception:
    _NC, _NS = 2, 16
_NW = _NC * _NS
_HALF = (HEADS * TOPK) // 2


_OBATCH = 8


def _bag_body(values_hbm, vi_hbm, w_hbm, out_hbm,
              vi_v, w_v, rows_v, out_v, idx0, idx1, sem0, sem1):
    ntok = vi_hbm.shape[0] // _NW
    wid = lax.axis_index("s") * _NC + lax.axis_index("c")
    base = pl.multiple_of(wid * ntok, ntok)
    pltpu.sync_copy(vi_hbm.at[pl.ds(base, ntok)], vi_v)
    pltpu.sync_copy(w_hbm.at[pl.ds(base, ntok)], w_v)
    idx_bufs = [idx0, idx1]

    def stage_idx(tok, half, buf):
        ib = idx_bufs[buf]
        ib[pl.ds(0, 16)] = vi_v[tok, pl.ds(half * _HALF, 16)]
        ib[pl.ds(16, 16)] = vi_v[tok, pl.ds(half * _HALF + 16, 16)]

    def copy(tok, half, buf, sem):
        return pltpu.make_async_copy(values_hbm.at[idx_bufs[buf]],
                                     rows_v.at[buf], sem)

    def accumulate(tok, half, buf, tm):
        for g in range(DIM // 128):
            gb = g * 128

            def rg_body(rg, accs):
                wv = w_v[tok, pl.ds(half * _HALF + rg * 16, 16)]
                for j in range(16):
                    wr = jnp.full((16,), wv[j], jnp.float32)
                    r = rg * 16 + j
                    accs = tuple(
                        accs[i] + rows_v[buf, r, pl.ds(gb + i * 16, 16)] * wr
                        for i in range(8))
                return accs

            accs = lax.fori_loop(
                0, _HALF // 16, rg_body,
                tuple(jnp.zeros((16,), jnp.float32) for _ in range(8)))
            for i in range(8):
                sl = pl.ds(gb + i * 16, 16)
                if half == 0:
                    out_v[tm, sl] = accs[i]
                else:
                    out_v[tm, sl] = out_v[tm, sl] + accs[i]

    stage_idx(0, 0, 0)
    copy(0, 0, 0, sem0).start()

    def tok_body(tok, _):
        tm = tok % _OBATCH
        stage_idx(tok, 1, 1)
        copy(tok, 1, 1, sem1).start()
        copy(tok, 0, 0, sem0).wait()
        accumulate(tok, 0, 0, tm)

        @pl.when(tok < ntok - 1)
        def _():
            stage_idx(tok + 1, 0, 0)
            copy(tok + 1, 0, 0, sem0).start()

        copy(tok, 1, 1, sem1).wait()
        accumulate(tok, 1, 1, tm)

        @pl.when(tm == _OBATCH - 1)
        def _():
            start = pl.multiple_of(base + tok - (_OBATCH - 1), _OBATCH)
            pltpu.sync_copy(out_v, out_hbm.at[pl.ds(start, _OBATCH)])
        return 0

    lax.fori_loop(0, ntok, tok_body, 0)


def _bag(values, vi, w, ntok):
    mesh = plsc.VectorSubcoreMesh(core_axis_name="c", subcore_axis_name="s")
    per = ntok // _NW
    kern = pl.kernel(
        _bag_body,
        out_type=jax.ShapeDtypeStruct((ntok, DIM), jnp.float32),
        mesh=mesh,
        scratch_types=[
            pltpu.VMEM((per, HEADS * TOPK), jnp.int32),
            pltpu.VMEM((per, HEADS * TOPK), jnp.float32),
            pltpu.VMEM((2, _HALF, DIM), jnp.float32),
            pltpu.VMEM((_OBATCH, DIM), jnp.float32),
            pltpu.VMEM((_HALF,), jnp.int32),
            pltpu.VMEM((_HALF,), jnp.int32),
            pltpu.SemaphoreType.DMA,
            pltpu.SemaphoreType.DMA,
        ],
    )
    return kern(values, vi, w)


def kernel(x, Wq, ln_gamma, ln_beta, keys_p, values):
    b, t, _ = x.shape
    xt = x.reshape(t, DIM).T
    kmat = jnp.transpose(keys_p, (2, 0, 1, 3)).reshape(
        NCHUNK, NUM_KEYS, DIM_HEAD)
    gamma = ln_gamma.reshape(DIM_HEAD, 1)
    beta = ln_beta.reshape(DIM_HEAD, 1)
    chunks = (256, 512, 512, 768)
    outs = []
    off = 0
    for tc in chunks:
        wt, vit = _route(xt[:, off:off + tc], Wq, gamma, beta, kmat)
        outs.append(_bag(values, vit.T, wt.T, tc))
        off += tc
    out = jnp.concatenate(outs, axis=0)
    return out.reshape(b, t, DIM)


# speedup vs baseline: 1.0969x; 1.0094x over previous
import functools

import jax
import jax.numpy as jnp
from jax import lax
from jax.experimental import pallas as pl
from jax.experimental.pallas import tpu as pltpu
from jax.experimental.pallas import tpu_sc as plsc

DIM = 1024
HEADS = 4
NUM_KEYS = 256
TOPK = 16
DIM_HEAD = 128
NCHUNK = 2 * HEADS

TOK_BLOCK = 256

_NEG = float("-inf")


def _topk_t(d):
    pos = lax.broadcasted_iota(jnp.int32, d.shape, 0)
    ms, js = [], []
    for _ in range(TOPK):
        m = jnp.max(d, axis=0, keepdims=True)
        j = jnp.min(jnp.where(d == m, pos, 1 << 30), axis=0, keepdims=True)
        ms.append(m)
        js.append(j)
        d = jnp.where(pos == j, _NEG, d)
    return jnp.concatenate(ms, axis=0), jnp.concatenate(js, axis=0)


def _route_body(xt_ref, wq_ref, g_ref, b_ref, keys_ref, w_ref, vi_ref):
    qt = jnp.dot(wq_ref[...], xt_ref[...],
                 preferred_element_type=jnp.float32)
    gamma = g_ref[...]
    beta = b_ref[...]

    def stage1(c):
        qc = qt[c * DIM_HEAD:(c + 1) * DIM_HEAD, :]
        mean = jnp.mean(qc, axis=0, keepdims=True)
        cen = qc - mean
        var = jnp.mean(cen * cen, axis=0, keepdims=True)
        qn = cen * lax.rsqrt(var + 1e-5) * gamma + beta
        d = jnp.dot(keys_ref[c], qn,
                    preferred_element_type=jnp.float32)
        return _topk_t(d)

    w_parts = []
    vi_parts = []
    for h in range(HEADS):
        sx, ixp = stage1(h)
        sy, iyp = stage1(HEADS + h)
        comb = jnp.concatenate(
            [sx[j:j + 1, :] + sy for j in range(TOPK)], axis=0)
        fs, fp = _topk_t(comb)
        jx = fp >> 4
        jy = fp & 15
        vix = jnp.zeros(fp.shape, jnp.int32)
        viy = jnp.zeros(fp.shape, jnp.int32)
        for cc in range(TOPK):
            vix = jnp.where(jx == cc, ixp[cc:cc + 1, :], vix)
            viy = jnp.where(jy == cc, iyp[cc:cc + 1, :], viy)
        e = jnp.exp(fs - fs[0:1, :])
        w_parts.append(e / jnp.sum(e, axis=0, keepdims=True))
        vi_parts.append(vix * NUM_KEYS + viy)
    w_ref[...] = jnp.concatenate(w_parts, axis=0).T
    vi_ref[...] = jnp.concatenate(vi_parts, axis=0).T


def _route(xt, wq, gamma, beta, kmat):
    t = xt.shape[1]
    grid = (t // TOK_BLOCK,)
    return pl.pallas_call(
        _route_body,
        grid=grid,
        in_specs=[
            pl.BlockSpec((DIM, TOK_BLOCK), lambda i: (0, i)),
            pl.BlockSpec((2 * HEADS * DIM_HEAD, DIM), lambda i: (0, 0)),
            pl.BlockSpec((DIM_HEAD, 1), lambda i: (0, 0)),
            pl.BlockSpec((DIM_HEAD, 1), lambda i: (0, 0)),
            pl.BlockSpec((NCHUNK, NUM_KEYS, DIM_HEAD), lambda i: (0, 0, 0)),
        ],
        out_specs=[
            pl.BlockSpec((TOK_BLOCK, HEADS * TOPK), lambda i: (i, 0)),
            pl.BlockSpec((TOK_BLOCK, HEADS * TOPK), lambda i: (i, 0)),
        ],
        out_shape=[
            jax.ShapeDtypeStruct((t, HEADS * TOPK), jnp.float32),
            jax.ShapeDtypeStruct((t, HEADS * TOPK), jnp.int32),
        ],
    )(xt, wq, gamma, beta, kmat)


try:
    _SC_INFO = plsc.get_sparse_core_info()
    _NC = _SC_INFO.num_cores
    _NS = _SC_INFO.num_subcores
except Exception:
    _NC, _NS = 2, 16
_NW = _NC * _NS
_HALF = (HEADS * TOPK) // 2


_OBATCH = 8


def _bag_body(values_hbm, vi_hbm, w_hbm, out_hbm,
              vi_v, w_v, rows_v, out_v, idx0, idx1, sem0, sem1):
    ntok = vi_hbm.shape[0] // _NW
    wid = lax.axis_index("s") * _NC + lax.axis_index("c")
    base = pl.multiple_of(wid * ntok, ntok)
    pltpu.sync_copy(vi_hbm.at[pl.ds(base, ntok)], vi_v)
    pltpu.sync_copy(w_hbm.at[pl.ds(base, ntok)], w_v)
    idx_bufs = [idx0, idx1]

    def stage_idx(tok, half, buf):
        ib = idx_bufs[buf]
        ib[pl.ds(0, 16)] = vi_v[tok, pl.ds(half * _HALF, 16)]
        ib[pl.ds(16, 16)] = vi_v[tok, pl.ds(half * _HALF + 16, 16)]

    def copy(tok, half, buf, sem):
        return pltpu.make_async_copy(values_hbm.at[idx_bufs[buf]],
                                     rows_v.at[buf], sem)

    def accumulate(tok, half, buf, tm):
        for g in range(DIM // 128):
            gb = g * 128

            def rg_body(rg, accs):
                wv = w_v[tok, pl.ds(half * _HALF + rg * 16, 16)]
                for j in range(16):
                    wr = jnp.full((16,), wv[j], jnp.float32)
                    r = rg * 16 + j
                    accs = tuple(
                        accs[i] + rows_v[buf, r, pl.ds(gb + i * 16, 16)] * wr
                        for i in range(8))
                return accs

            accs = lax.fori_loop(
                0, _HALF // 16, rg_body,
                tuple(jnp.zeros((16,), jnp.float32) for _ in range(8)))
            for i in range(8):
                sl = pl.ds(gb + i * 16, 16)
                if half == 0:
                    out_v[tm, sl] = accs[i]
                else:
                    out_v[tm, sl] = out_v[tm, sl] + accs[i]

    stage_idx(0, 0, 0)
    copy(0, 0, 0, sem0).start()

    def tok_body(tok, _):
        tm = tok % _OBATCH
        stage_idx(tok, 1, 1)
        copy(tok, 1, 1, sem1).start()
        copy(tok, 0, 0, sem0).wait()
        accumulate(tok, 0, 0, tm)

        @pl.when(tok < ntok - 1)
        def _():
            stage_idx(tok + 1, 0, 0)
            copy(tok + 1, 0, 0, sem0).start()

        copy(tok, 1, 1, sem1).wait()
        accumulate(tok, 1, 1, tm)

        @pl.when(tm == _OBATCH - 1)
        def _():
            start = pl.multiple_of(base + tok - (_OBATCH - 1), _OBATCH)
            pltpu.sync_copy(out_v, out_hbm.at[pl.ds(start, _OBATCH)])
        return 0

    lax.fori_loop(0, ntok, tok_body, 0)


def _bag(values, vi, w, ntok):
    mesh = plsc.VectorSubcoreMesh(core_axis_name="c", subcore_axis_name="s")
    per = ntok // _NW
    kern = pl.kernel(
        _bag_body,
        out_type=jax.ShapeDtypeStruct((ntok, DIM), jnp.float32),
        mesh=mesh,
        scratch_types=[
            pltpu.VMEM((per, HEADS * TOPK), jnp.int32),
            pltpu.VMEM((per, HEADS * TOPK), jnp.float32),
            pltpu.VMEM((2, _HALF, DIM), jnp.float32),
            pltpu.VMEM((_OBATCH, DIM), jnp.float32),
            pltpu.VMEM((_HALF,), jnp.int32),
            pltpu.VMEM((_HALF,), jnp.int32),
            pltpu.SemaphoreType.DMA,
            pltpu.SemaphoreType.DMA,
        ],
    )
    return kern(values, vi, w)


def kernel(x, Wq, ln_gamma, ln_beta, keys_p, values):
    b, t, _ = x.shape
    xt = x.reshape(t, DIM).T
    kmat = jnp.transpose(keys_p, (2, 0, 1, 3)).reshape(
        NCHUNK, NUM_KEYS, DIM_HEAD)
    gamma = ln_gamma.reshape(DIM_HEAD, 1)
    beta = ln_beta.reshape(DIM_HEAD, 1)
    chunks = (256, 512, 512, 768)
    outs = []
    off = 0
    for tc in chunks:
        w, vi = _route(xt[:, off:off + tc], Wq, gamma, beta, kmat)
        outs.append(_bag(values, vi, w, tc))
        off += tc
    out = jnp.concatenate(outs, axis=0)
    return out.reshape(b, t, DIM)


# speedup vs baseline: 1.1512x; 1.0495x over previous
import functools

import jax
import jax.numpy as jnp
from jax import lax
from jax.experimental import pallas as pl
from jax.experimental.pallas import tpu as pltpu
from jax.experimental.pallas import tpu_sc as plsc

DIM = 1024
HEADS = 4
NUM_KEYS = 256
TOPK = 16
DIM_HEAD = 128
NCHUNK = 2 * HEADS

TOK_BLOCK = 256

_NEG = float("-inf")


def _topk_t(d):
    pos = lax.broadcasted_iota(jnp.int32, d.shape, 0)
    ms, js = [], []
    for _ in range(TOPK):
        m = jnp.max(d, axis=0, keepdims=True)
        j = jnp.min(jnp.where(d == m, pos, 1 << 30), axis=0, keepdims=True)
        ms.append(m)
        js.append(j)
        d = jnp.where(pos == j, _NEG, d)
    return jnp.concatenate(ms, axis=0), jnp.concatenate(js, axis=0)


def _route_body(x_ref, wq_ref, g_ref, b_ref, keys_ref, w_ref, vi_ref):
    qt = lax.dot_general(wq_ref[...], x_ref[...],
                         (((1,), (1,)), ((), ())),
                         preferred_element_type=jnp.float32)
    gamma = g_ref[...]
    beta = b_ref[...]

    def stage1(c):
        qc = qt[c * DIM_HEAD:(c + 1) * DIM_HEAD, :]
        mean = jnp.mean(qc, axis=0, keepdims=True)
        cen = qc - mean
        var = jnp.mean(cen * cen, axis=0, keepdims=True)
        qn = cen * lax.rsqrt(var + 1e-5) * gamma + beta
        d = jnp.dot(keys_ref[c], qn,
                    preferred_element_type=jnp.float32)
        return _topk_t(d)

    w_parts = []
    vi_parts = []
    for h in range(HEADS):
        sx, ixp = stage1(h)
        sy, iyp = stage1(HEADS + h)
        comb = jnp.concatenate(
            [sx[j:j + 1, :] + sy for j in range(TOPK)], axis=0)
        fs, fp = _topk_t(comb)
        jx = fp >> 4
        jy = fp & 15
        vix = jnp.zeros(fp.shape, jnp.int32)
        viy = jnp.zeros(fp.shape, jnp.int32)
        for cc in range(TOPK):
            vix = jnp.where(jx == cc, ixp[cc:cc + 1, :], vix)
            viy = jnp.where(jy == cc, iyp[cc:cc + 1, :], viy)
        e = jnp.exp(fs - fs[0:1, :])
        w_parts.append(e / jnp.sum(e, axis=0, keepdims=True))
        vi_parts.append(vix * NUM_KEYS + viy)
    w_ref[...] = jnp.concatenate(w_parts, axis=0).T
    vi_ref[...] = jnp.concatenate(vi_parts, axis=0).T


def _route(x2d, wq, gamma, beta, kmat):
    t = x2d.shape[0]
    grid = (t // TOK_BLOCK,)
    return pl.pallas_call(
        _route_body,
        grid=grid,
        in_specs=[
            pl.BlockSpec((TOK_BLOCK, DIM), lambda i: (i, 0)),
            pl.BlockSpec((2 * HEADS * DIM_HEAD, DIM), lambda i: (0, 0)),
            pl.BlockSpec((DIM_HEAD, 1), lambda i: (0, 0)),
            pl.BlockSpec((DIM_HEAD, 1), lambda i: (0, 0)),
            pl.BlockSpec((NCHUNK, NUM_KEYS, DIM_HEAD), lambda i: (0, 0, 0)),
        ],
        out_specs=[
            pl.BlockSpec((TOK_BLOCK, HEADS * TOPK), lambda i: (i, 0)),
            pl.BlockSpec((TOK_BLOCK, HEADS * TOPK), lambda i: (i, 0)),
        ],
        out_shape=[
            jax.ShapeDtypeStruct((t, HEADS * TOPK), jnp.float32),
            jax.ShapeDtypeStruct((t, HEADS * TOPK), jnp.int32),
        ],
    )(x2d, wq, gamma, beta, kmat)


try:
    _SC_INFO = plsc.get_sparse_core_info()
    _NC = _SC_INFO.num_cores
    _NS = _SC_INFO.num_subcores
except Exception:
    _NC, _NS = 2, 16
_NW = _NC * _NS
_HALF = (HEADS * TOPK) // 2


_OBATCH = 8


def _bag_body(values_hbm, vi_hbm, w_hbm, out_hbm,
              vi_v, w_v, rows_v, out_v, idx0, idx1, sem0, sem1):
    ntok = vi_hbm.shape[0] // _NW
    wid = lax.axis_index("s") * _NC + lax.axis_index("c")
    base = pl.multiple_of(wid * ntok, ntok)
    pltpu.sync_copy(vi_hbm.at[pl.ds(base, ntok)], vi_v)
    pltpu.sync_copy(w_hbm.at[pl.ds(base, ntok)], w_v)
    idx_bufs = [idx0, idx1]

    def stage_idx(tok, half, buf):
        ib = idx_bufs[buf]
        ib[pl.ds(0, 16)] = vi_v[tok, pl.ds(half * _HALF, 16)]
        ib[pl.ds(16, 16)] = vi_v[tok, pl.ds(half * _HALF + 16, 16)]

    def copy(tok, half, buf, sem):
        return pltpu.make_async_copy(values_hbm.at[idx_bufs[buf]],
                                     rows_v.at[buf], sem)

    def accumulate(tok, half, buf, tm):
        for g in range(DIM // 128):
            gb = g * 128

            def rg_body(rg, accs):
                wv = w_v[tok, pl.ds(half * _HALF + rg * 16, 16)]
                for j in range(16):
                    wr = jnp.full((16,), wv[j], jnp.float32)
                    r = rg * 16 + j
                    accs = tuple(
                        accs[i] + rows_v[buf, r, pl.ds(gb + i * 16, 16)] * wr
                        for i in range(8))
                return accs

            accs = lax.fori_loop(
                0, _HALF // 16, rg_body,
                tuple(jnp.zeros((16,), jnp.float32) for _ in range(8)))
            for i in range(8):
                sl = pl.ds(gb + i * 16, 16)
                if half == 0:
                    out_v[tm, sl] = accs[i]
                else:
                    out_v[tm, sl] = out_v[tm, sl] + accs[i]

    stage_idx(0, 0, 0)
    copy(0, 0, 0, sem0).start()

    def tok_body(tok, _):
        tm = tok % _OBATCH
        stage_idx(tok, 1, 1)
        copy(tok, 1, 1, sem1).start()
        copy(tok, 0, 0, sem0).wait()
        accumulate(tok, 0, 0, tm)

        @pl.when(tok < ntok - 1)
        def _():
            stage_idx(tok + 1, 0, 0)
            copy(tok + 1, 0, 0, sem0).start()

        copy(tok, 1, 1, sem1).wait()
        accumulate(tok, 1, 1, tm)

        @pl.when(tm == _OBATCH - 1)
        def _():
            start = pl.multiple_of(base + tok - (_OBATCH - 1), _OBATCH)
            pltpu.sync_copy(out_v, out_hbm.at[pl.ds(start, _OBATCH)])
        return 0

    lax.fori_loop(0, ntok, tok_body, 0)


def _bag(values, vi, w, ntok):
    mesh = plsc.VectorSubcoreMesh(core_axis_name="c", subcore_axis_name="s")
    per = ntok // _NW
    kern = pl.kernel(
        _bag_body,
        out_type=jax.ShapeDtypeStruct((ntok, DIM), jnp.float32),
        mesh=mesh,
        scratch_types=[
            pltpu.VMEM((per, HEADS * TOPK), jnp.int32),
            pltpu.VMEM((per, HEADS * TOPK), jnp.float32),
            pltpu.VMEM((2, _HALF, DIM), jnp.float32),
            pltpu.VMEM((_OBATCH, DIM), jnp.float32),
            pltpu.VMEM((_HALF,), jnp.int32),
            pltpu.VMEM((_HALF,), jnp.int32),
            pltpu.SemaphoreType.DMA,
            pltpu.SemaphoreType.DMA,
        ],
    )
    return kern(values, vi, w)


def kernel(x, Wq, ln_gamma, ln_beta, keys_p, values):
    b, t, _ = x.shape
    x2d = x.reshape(t, DIM)
    kmat = jnp.transpose(keys_p, (2, 0, 1, 3)).reshape(
        NCHUNK, NUM_KEYS, DIM_HEAD)
    gamma = ln_gamma.reshape(DIM_HEAD, 1)
    beta = ln_beta.reshape(DIM_HEAD, 1)
    chunks = (256, 512, 512, 768)
    outs = []
    off = 0
    for tc in chunks:
        w, vi = _route(x2d[off:off + tc], Wq, gamma, beta, kmat)
        outs.append(_bag(values, vi, w, tc))
        off += tc
    out = jnp.concatenate(outs, axis=0)
    return out.reshape(b, t, DIM)
